# baseline (speedup 1.0000x reference)
import functools

import jax
import jax.numpy as jnp
from jax import lax
from jax.experimental import pallas as pl
from jax.experimental.pallas import tpu as pltpu
from jax.experimental.pallas import tpu_sc as plsc

D_MODEL = 64
NUM_CORES = 2
NUM_SUBCORES = 16
NUM_WORKERS = NUM_CORES * NUM_SUBCORES
NBUF = 4


@jax.jit
def _gather_rows(x, table):
    n_b, n_s = x.shape
    b_per_w = n_b // NUM_WORKERS
    assert b_per_w % NBUF == 0
    mesh = plsc.VectorSubcoreMesh(core_axis_name="c", subcore_axis_name="s")

    @functools.partial(
        pl.kernel,
        mesh=mesh,
        out_type=jax.ShapeDtypeStruct((n_b, n_s, D_MODEL), jnp.float32),
        scratch_types=[
            pltpu.VMEM((b_per_w, n_s), jnp.int32),
            pltpu.VMEM((NBUF, n_s, D_MODEL), jnp.float32),
            pltpu.SemaphoreType.DMA,
            pltpu.SemaphoreType.DMA,
        ],
        compiler_params=pltpu.CompilerParams(use_tc_tiling_on_sc=False),
    )
    def k(x_hbm, table_hbm, out_hbm, idx_v, rows_v, g_sem, o_sem):
        wid = lax.axis_index("s") * NUM_CORES + lax.axis_index("c")
        base = wid * b_per_w
        pltpu.sync_copy(x_hbm.at[pl.ds(base, b_per_w)], idx_v)

        def gather(r, s):
            pltpu.async_copy(table_hbm.at[idx_v.at[r]], rows_v.at[s], g_sem)

        def out_copy(r, s):
            pltpu.async_copy(rows_v.at[s], out_hbm.at[base + r], o_sem)

        def wait_out(s):
            pltpu.make_async_copy(rows_v.at[s], out_hbm.at[base], o_sem).wait()

        def wait_gather(s):
            pltpu.make_async_copy(
                table_hbm.at[idx_v.at[0]], rows_v.at[s], g_sem
            ).wait()

        def step(r, j):
            prev = (j - 1) % NBUF

            @pl.when(r >= 1)
            def _():
                wait_out(prev)

            @pl.when(r + NBUF - 1 < b_per_w)
            def _():
                gather(r + NBUF - 1, prev)

            wait_gather(j)
            out_copy(r, j)

        for s0 in range(NBUF - 1):
            gather(s0, s0)

        def body(p, carry):
            for j in range(NBUF):
                step(NBUF * p + j, j)
            return carry

        lax.fori_loop(0, b_per_w // NBUF, body, 0)
        wait_out((b_per_w - 1) % NBUF)

    return k(x, table)


def kernel(x, table):
    return _gather_rows(x.astype(jnp.int32), table)


# speedup vs baseline: 1.0175x; 1.0175x over previous
import functools

import jax
import jax.numpy as jnp
from jax import lax
from jax.experimental import pallas as pl
from jax.experimental.pallas import tpu as pltpu
from jax.experimental.pallas import tpu_sc as plsc

D_MODEL = 64
NUM_CORES = 2
NUM_SUBCORES = 16
NUM_WORKERS = NUM_CORES * NUM_SUBCORES
CHUNK = 640


@functools.partial(jax.jit, static_argnums=(2,))
def _gather_rows(idx, table, n_rows):
    n_per_w = n_rows // NUM_WORKERS
    n_chunks = n_per_w // CHUNK
    assert n_chunks % 2 == 0
    idx3 = idx.reshape(NUM_WORKERS, n_chunks, CHUNK)
    mesh = plsc.VectorSubcoreMesh(core_axis_name="c", subcore_axis_name="s")

    @functools.partial(
        pl.kernel,
        mesh=mesh,
        out_type=jax.ShapeDtypeStruct((n_rows, D_MODEL), jnp.float32),
        scratch_types=[
            pltpu.VMEM((n_chunks, CHUNK), jnp.int32),
            pltpu.VMEM((2, CHUNK, D_MODEL), jnp.float32),
            pltpu.SemaphoreType.DMA,
            pltpu.SemaphoreType.DMA,
        ],
        compiler_params=pltpu.CompilerParams(use_tc_tiling_on_sc=False),
    )
    def k(idx_hbm, table_hbm, out_hbm, idx_v, rows_v, g_sem, o_sem):
        wid = lax.axis_index("s") * NUM_CORES + lax.axis_index("c")
        base = wid * n_per_w
        pltpu.sync_copy(idx_hbm.at[wid], idx_v)
        pltpu.async_copy(table_hbm.at[idx_v.at[0]], rows_v.at[0], g_sem)

        def step(i, s, s_next):
            @pl.when(i >= 1)
            def _():
                pltpu.make_async_copy(
                    rows_v.at[s_next],
                    out_hbm.at[pl.ds(base, CHUNK)],
                    o_sem,
                ).wait()

            @pl.when(i + 1 < n_chunks)
            def _():
                pltpu.async_copy(
                    table_hbm.at[idx_v.at[i + 1]], rows_v.at[s_next], g_sem
                )

            pltpu.make_async_copy(
                table_hbm.at[idx_v.at[i]], rows_v.at[s], g_sem
            ).wait()
            pltpu.async_copy(
                rows_v.at[s], out_hbm.at[pl.ds(base + i * CHUNK, CHUNK)], o_sem
            )

        def body(p, carry):
            step(2 * p, 0, 1)
            step(2 * p + 1, 1, 0)
            return carry

        lax.fori_loop(0, n_chunks // 2, body, 0)
        pltpu.make_async_copy(
            rows_v.at[1], out_hbm.at[pl.ds(base, CHUNK)], o_sem
        ).wait()

    return k(idx3, table)


def kernel(x, table):
    b, s = x.shape
    n_rows = b * s
    idx = x.reshape(n_rows).astype(jnp.int32)
    out = _gather_rows(idx, table, n_rows)
    return out.reshape(b, s, D_MODEL)
